# baseline (speedup 1.0000x reference)
import jax
import jax.numpy as jnp
from jax import lax
from jax.experimental import pallas as pl
from jax.experimental.pallas import tpu as pltpu

B, L, D = 16, 2048, 1024
CH = 256
NCH = L // CH


def _body(len_ref, blk, out_ref):
    i = pl.program_id(0)
    c = pl.program_id(1)
    n = len_ref[i]
    rv = n - c * CH

    @pl.when(rv > 0)
    def _():
        def full_sum(_):
            return jnp.sum(blk[0], axis=0)

        def masked_sum(_):
            row_id = lax.broadcasted_iota(jnp.int32, (CH, 1), 0)
            w = (row_id < rv).astype(jnp.float32)
            return jnp.sum(blk[0] * w, axis=0)

        s = lax.cond(rv >= CH, full_sum, masked_sum, 0)

        @pl.when(c == 0)
        def _():
            out_ref[i, :] = s

        @pl.when(c > 0)
        def _():
            out_ref[i, :] = out_ref[i, :] + s

    @pl.when(c == NCH - 1)
    def _():
        out_ref[i, :] = out_ref[i, :] / n.astype(jnp.float32)


def kernel(input, length):
    n = length.astype(jnp.int32)

    def in_map(i, c, len_r):
        cap = lax.div(len_r[i] - 1, CH)
        return (i, jnp.minimum(c, cap), 0)

    grid_spec = pltpu.PrefetchScalarGridSpec(
        num_scalar_prefetch=1,
        grid=(B, NCH),
        in_specs=[pl.BlockSpec((1, CH, D), in_map)],
        out_specs=pl.BlockSpec((B, D), lambda i, c, len_r: (0, 0)),
    )
    return pl.pallas_call(
        _body,
        grid_spec=grid_spec,
        out_shape=jax.ShapeDtypeStruct((B, D), jnp.float32),
    )(n, input)


# speedup vs baseline: 2.3553x; 2.3553x over previous
import jax
import jax.numpy as jnp
from jax import lax
from jax.experimental import pallas as pl
from jax.experimental.pallas import tpu as pltpu

B, L, D = 16, 2048, 1024
CH = 256
NCH = L // CH


def _body(len_ref, in_hbm, out_ref, buf, sem):
    i = pl.program_id(0)
    n = len_ref[i]
    slot = lax.rem(i, 2)

    def mk(idx, sl, k):
        return pltpu.make_async_copy(
            in_hbm.at[idx, pl.ds(0, k * CH), :],
            buf.at[sl, pl.ds(0, k * CH), :],
            sem.at[sl],
        )

    def issue(idx, sl):
        kk = lax.div(len_ref[idx] - 1, CH)
        lax.switch(kk, [lambda k=k: mk(idx, sl, k + 1).start()
                        for k in range(NCH)])

    def wait(idx, sl):
        kk = lax.div(len_ref[idx] - 1, CH)
        lax.switch(kk, [lambda k=k: mk(idx, sl, k + 1).wait()
                        for k in range(NCH)])

    @pl.when(i == 0)
    def _():
        issue(0, 0)

    @pl.when(i + 1 < B)
    def _():
        issue(i + 1, lax.rem(i + 1, 2))

    wait(i, slot)

    nch = lax.div(n - 1, CH) + 1

    def step(c, acc):
        rv = n - c * CH

        def full_sum(_):
            return jnp.sum(buf[slot, pl.ds(c * CH, CH), :], axis=0)

        def masked_sum(_):
            row_id = lax.broadcasted_iota(jnp.int32, (CH, 1), 0)
            w = (row_id < rv).astype(jnp.float32)
            return jnp.sum(buf[slot, pl.ds(c * CH, CH), :] * w, axis=0)

        return acc + lax.cond(rv >= CH, full_sum, masked_sum, 0)

    acc = lax.fori_loop(0, nch, step, jnp.zeros((D,), jnp.float32))
    out_ref[i, :] = acc / n.astype(jnp.float32)


def kernel(input, length):
    n = length.astype(jnp.int32)
    grid_spec = pltpu.PrefetchScalarGridSpec(
        num_scalar_prefetch=1,
        grid=(B,),
        in_specs=[pl.BlockSpec(memory_space=pl.ANY)],
        out_specs=pl.BlockSpec((B, D), lambda i, len_r: (0, 0)),
        scratch_shapes=[
            pltpu.VMEM((2, L, D), jnp.float32),
            pltpu.SemaphoreType.DMA((2,)),
        ],
    )
    return pl.pallas_call(
        _body,
        grid_spec=grid_spec,
        out_shape=jax.ShapeDtypeStruct((B, D), jnp.float32),
    )(n, input)


# speedup vs baseline: 2.4026x; 1.0201x over previous
import jax
import jax.numpy as jnp
from jax import lax
from jax.experimental import pallas as pl
from jax.experimental.pallas import tpu as pltpu

B, L, D = 16, 2048, 1024
CH = 128
NCH = L // CH


def _body(len_ref, in_hbm, out_ref, buf, sem):
    i = pl.program_id(0)
    n = len_ref[i]
    slot = lax.rem(i, 2)

    def mk(idx, sl, k):
        return pltpu.make_async_copy(
            in_hbm.at[idx, pl.ds(0, k * CH), :],
            buf.at[sl, pl.ds(0, k * CH), :],
            sem.at[sl],
        )

    def issue(idx, sl):
        kk = lax.div(len_ref[idx] - 1, CH)
        lax.switch(kk, [lambda k=k: mk(idx, sl, k + 1).start()
                        for k in range(NCH)])

    def wait(idx, sl):
        kk = lax.div(len_ref[idx] - 1, CH)
        lax.switch(kk, [lambda k=k: mk(idx, sl, k + 1).wait()
                        for k in range(NCH)])

    @pl.when(i == 0)
    def _():
        issue(0, 0)

    @pl.when(i + 1 < B)
    def _():
        issue(i + 1, lax.rem(i + 1, 2))

    wait(i, slot)

    nch = lax.div(n - 1, CH) + 1

    def step(c, acc):
        rv = n - c * CH

        def full_sum(_):
            return jnp.sum(buf[slot, pl.ds(c * CH, CH), :], axis=0)

        def masked_sum(_):
            row_id = lax.broadcasted_iota(jnp.int32, (CH, 1), 0)
            w = (row_id < rv).astype(jnp.float32)
            return jnp.sum(buf[slot, pl.ds(c * CH, CH), :] * w, axis=0)

        return acc + lax.cond(rv >= CH, full_sum, masked_sum, 0)

    acc = lax.fori_loop(0, nch, step, jnp.zeros((D,), jnp.float32))
    out_ref[i, :] = acc / n.astype(jnp.float32)


def kernel(input, length):
    n = length.astype(jnp.int32)
    grid_spec = pltpu.PrefetchScalarGridSpec(
        num_scalar_prefetch=1,
        grid=(B,),
        in_specs=[pl.BlockSpec(memory_space=pl.ANY)],
        out_specs=pl.BlockSpec((B, D), lambda i, len_r: (0, 0)),
        scratch_shapes=[
            pltpu.VMEM((2, L, D), jnp.float32),
            pltpu.SemaphoreType.DMA((2,)),
        ],
    )
    return pl.pallas_call(
        _body,
        grid_spec=grid_spec,
        out_shape=jax.ShapeDtypeStruct((B, D), jnp.float32),
    )(n, input)
